# baseline (speedup 1.0000x reference)
import functools

import jax
import jax.numpy as jnp
from jax import lax
from jax.experimental import pallas as pl
from jax.experimental.pallas import tpu as pltpu
from jax.experimental.pallas import tpu_sc as plsc

NC = 2
NS = 16
NW = NC * NS


def _linearize_tc(table):
    V, D = table.shape
    R = 2048

    def body(in_ref, out_ref):
        out_ref[:, 0:D] = in_ref[...]

    return pl.pallas_call(
        body,
        grid=(V // R,),
        in_specs=[pl.BlockSpec((R, D), lambda i: (i, 0))],
        out_specs=pl.BlockSpec((R, 2 * D), lambda i: (i, 0)),
        out_shape=jax.ShapeDtypeStruct((V, 2 * D), jnp.float32),
    )(table)


def _sc_gather(table, ctx_idx, q_idx, B, CL, QL, QLP):
    V, DT = table.shape
    D = DT // 2
    DP = DT
    b_per_w = B // NW
    ctx_per_w = b_per_w * CL
    q_per_w = b_per_w * QL

    mesh = plsc.VectorSubcoreMesh(core_axis_name="c", subcore_axis_name="s")

    @functools.partial(
        pl.kernel,
        mesh=mesh,
        compiler_params=pltpu.CompilerParams(use_tc_tiling_on_sc=False),
        out_type=(
            jax.ShapeDtypeStruct((B, CL, DP), jnp.float32),
            jax.ShapeDtypeStruct((B, QLP, DP), jnp.float32),
        ),
        scratch_types=[
            pltpu.VMEM((ctx_per_w,), jnp.int32),
            pltpu.VMEM((CL, DT), jnp.float32),
            pltpu.VMEM((CL, DT), jnp.float32),
            pltpu.SemaphoreType.DMA,
            pltpu.SemaphoreType.DMA,
            pltpu.SemaphoreType.DMA,
            pltpu.SemaphoreType.DMA,
        ],
    )
    def k(table_hbm, ctx_idx_hbm, q_idx_hbm, ctx_out, q_out,
          idx_v, rows0, rows1, sg0, sg1, sw0, sw1):
        wid = lax.axis_index("s") * NC + lax.axis_index("c")
        b_base = wid * b_per_w

        def pipe(idx_hbm, out_hbm, per_w, L, rows_per_chunk):
            C = L * rows_per_chunk
            n = b_per_w // rows_per_chunk
            base = wid * per_w
            pltpu.sync_copy(idx_hbm.at[pl.ds(base, per_w)],
                            idx_v.at[pl.ds(0, per_w)])
            bufs = ((rows0, sg0, sw0), (rows1, sg1, sw1))

            def start_gather(g, rows, sg):
                pltpu.async_copy(
                    table_hbm.at[idx_v.at[pl.ds(g * C, C)]],
                    rows.at[pl.ds(0, C)], sg)

            def wait_gather(rows, sg):
                pltpu.make_async_copy(
                    table_hbm.at[idx_v.at[pl.ds(0, C)]],
                    rows.at[pl.ds(0, C)], sg).wait()

            def start_write(g, rows, sw):
                for r in range(rows_per_chunk):
                    pltpu.async_copy(
                        rows.at[pl.ds(r * L, L)].at[:, pl.ds(0, D)],
                        out_hbm.at[b_base + g * rows_per_chunk + r]
                               .at[pl.ds(0, L), pl.ds(0, D)], sw)

            def wait_write(rows, sw):
                for r in range(rows_per_chunk):
                    pltpu.make_async_copy(
                        rows.at[pl.ds(r * L, L)].at[:, pl.ds(0, D)],
                        out_hbm.at[b_base].at[pl.ds(0, L), pl.ds(0, D)],
                        sw).wait()

            start_gather(0, rows0, sg0)
            start_gather(1, rows1, sg1)

            @pl.loop(0, n, step=2)
            def _(g):
                for j, (rows, sg, sw) in enumerate(bufs):
                    gg = g + j
                    wait_gather(rows, sg)
                    start_write(gg, rows, sw)

                    @pl.when(gg + 2 < n)
                    def _():
                        wait_write(rows, sw)
                        start_gather(gg + 2, rows, sg)

            wait_write(rows0, sw0)
            wait_write(rows1, sw1)

        pipe(ctx_idx_hbm, ctx_out, ctx_per_w, CL, 1)
        pipe(q_idx_hbm, q_out, q_per_w, QL, 2)

    return k(table, ctx_idx, q_idx)


def kernel(word_embeddings, input_context, input_question):
    B, CL = input_context.shape
    _, QL = input_question.shape
    D = word_embeddings.shape[1]
    QLP = (QL + 7) // 8 * 8
    ctx_idx = input_context.reshape(-1).astype(jnp.int32)
    q_idx = input_question.reshape(-1).astype(jnp.int32)
    table_lin = _linearize_tc(word_embeddings)
    ctx_pad, q_pad = _sc_gather(table_lin, ctx_idx, q_idx,
                                B, CL, QL, QLP)
    return (ctx_pad[:, :, :D], q_pad[:, :QL, :D])


# speedup vs baseline: 1.0022x; 1.0022x over previous
import functools

import jax
import jax.numpy as jnp
from jax import lax
from jax.experimental import pallas as pl
from jax.experimental.pallas import tpu as pltpu
from jax.experimental.pallas import tpu_sc as plsc

NC = 2
NS = 16
NW = NC * NS


def _linearize_tc(table):
    V, D = table.shape
    R = 2000
    assert V % R == 0

    def body(in_ref, out_ref):
        out_ref[:, 0:D] = in_ref[...]

    return pl.pallas_call(
        body,
        grid=(V // R,),
        in_specs=[pl.BlockSpec((R, D), lambda i: (i, 0))],
        out_specs=pl.BlockSpec((R, 2 * D), lambda i: (i, 0)),
        out_shape=jax.ShapeDtypeStruct((V, 2 * D), jnp.float32),
    )(table)


def _sc_gather(table, ctx_idx, q_idx, B, CL, QL, QLP):
    V, DT = table.shape
    D = DT // 2
    DP = DT
    b_per_w = B // NW
    ctx_per_w = b_per_w * CL
    q_per_w = b_per_w * QL

    mesh = plsc.VectorSubcoreMesh(core_axis_name="c", subcore_axis_name="s")

    @functools.partial(
        pl.kernel,
        mesh=mesh,
        compiler_params=pltpu.CompilerParams(use_tc_tiling_on_sc=False),
        out_type=(
            jax.ShapeDtypeStruct((B, CL, DP), jnp.float32),
            jax.ShapeDtypeStruct((B, QLP, DP), jnp.float32),
        ),
        scratch_types=[
            pltpu.VMEM((ctx_per_w,), jnp.int32),
            pltpu.VMEM((CL, DT), jnp.float32),
            pltpu.VMEM((CL, DT), jnp.float32),
            pltpu.SemaphoreType.DMA,
            pltpu.SemaphoreType.DMA,
            pltpu.SemaphoreType.DMA,
            pltpu.SemaphoreType.DMA,
        ],
    )
    def k(table_hbm, ctx_idx_hbm, q_idx_hbm, ctx_out, q_out,
          idx_v, rows0, rows1, sg0, sg1, sw0, sw1):
        wid = lax.axis_index("s") * NC + lax.axis_index("c")
        b_base = wid * b_per_w

        def pipe(idx_hbm, out_hbm, per_w, L, rows_per_chunk):
            C = L * rows_per_chunk
            n = b_per_w // rows_per_chunk
            base = wid * per_w
            pltpu.sync_copy(idx_hbm.at[pl.ds(base, per_w)],
                            idx_v.at[pl.ds(0, per_w)])
            bufs = ((rows0, sg0, sw0), (rows1, sg1, sw1))

            def start_gather(g, rows, sg):
                pltpu.async_copy(
                    table_hbm.at[idx_v.at[pl.ds(g * C, C)]],
                    rows.at[pl.ds(0, C)], sg)

            def wait_gather(rows, sg):
                pltpu.make_async_copy(
                    table_hbm.at[idx_v.at[pl.ds(0, C)]],
                    rows.at[pl.ds(0, C)], sg).wait()

            def start_write(g, rows, sw):
                for r in range(rows_per_chunk):
                    pltpu.async_copy(
                        rows.at[pl.ds(r * L, L)].at[:, pl.ds(0, D)],
                        out_hbm.at[b_base + g * rows_per_chunk + r]
                               .at[pl.ds(0, L), pl.ds(0, D)], sw)

            def wait_write(rows, sw):
                for r in range(rows_per_chunk):
                    pltpu.make_async_copy(
                        rows.at[pl.ds(r * L, L)].at[:, pl.ds(0, D)],
                        out_hbm.at[b_base].at[pl.ds(0, L), pl.ds(0, D)],
                        sw).wait()

            start_gather(0, rows0, sg0)
            start_gather(1, rows1, sg1)

            @pl.loop(0, n, step=2)
            def _(g):
                for j, (rows, sg, sw) in enumerate(bufs):
                    gg = g + j
                    wait_gather(rows, sg)
                    start_write(gg, rows, sw)

                    @pl.when(gg + 2 < n)
                    def _():
                        wait_write(rows, sw)
                        start_gather(gg + 2, rows, sg)

            wait_write(rows0, sw0)
            wait_write(rows1, sw1)

        pipe(ctx_idx_hbm, ctx_out, ctx_per_w, CL, 1)
        pipe(q_idx_hbm, q_out, q_per_w, QL, 2)

    return k(table, ctx_idx, q_idx)


def kernel(word_embeddings, input_context, input_question):
    B, CL = input_context.shape
    _, QL = input_question.shape
    D = word_embeddings.shape[1]
    QLP = (QL + 7) // 8 * 8
    ctx_idx = input_context.reshape(-1).astype(jnp.int32)
    q_idx = input_question.reshape(-1).astype(jnp.int32)
    table_lin = _linearize_tc(word_embeddings)
    ctx_pad, q_pad = _sc_gather(table_lin, ctx_idx, q_idx,
                                B, CL, QL, QLP)
    return (ctx_pad[:, :, :D], q_pad[:, :QL, :D])


# speedup vs baseline: 1.3324x; 1.3294x over previous
import functools

import jax
import jax.numpy as jnp
from jax import lax
from jax.experimental import pallas as pl
from jax.experimental.pallas import tpu as pltpu
from jax.experimental.pallas import tpu_sc as plsc

NC = 2
NS = 16
NW = NC * NS


def _sc_gather(table, ctx_idx, q_idx, B, CL, QL, QLP):
    V, D = table.shape
    DP = 2 * D
    b_per_w = B // NW
    ctx_per_w = b_per_w * CL
    q_per_w = b_per_w * QL

    mesh = plsc.VectorSubcoreMesh(core_axis_name="c", subcore_axis_name="s")

    @functools.partial(
        pl.kernel,
        mesh=mesh,
        compiler_params=pltpu.CompilerParams(use_tc_tiling_on_sc=False),
        out_type=(
            jax.ShapeDtypeStruct((B, CL, DP), jnp.float32),
            jax.ShapeDtypeStruct((B, QLP, DP), jnp.float32),
        ),
        scratch_types=[
            pltpu.VMEM((ctx_per_w,), jnp.int32),
            pltpu.VMEM((2 * CL, D), jnp.float32),
            pltpu.VMEM((2 * CL, D), jnp.float32),
            pltpu.SemaphoreType.DMA,
            pltpu.SemaphoreType.DMA,
            pltpu.SemaphoreType.DMA,
            pltpu.SemaphoreType.DMA,
        ],
    )
    def k(table_hbm, ctx_idx_hbm, q_idx_hbm, ctx_out, q_out,
          idx_v, rows0, rows1, sg0, sg1, sw0, sw1):
        wid = lax.axis_index("s") * NC + lax.axis_index("c")
        b_base = wid * b_per_w

        def pipe(idx_hbm, out_hbm, per_w, L, rows_per_chunk):
            C = L * rows_per_chunk
            n = b_per_w // rows_per_chunk
            base = wid * per_w
            pltpu.sync_copy(idx_hbm.at[pl.ds(base, per_w)],
                            idx_v.at[pl.ds(0, per_w)])
            bufs = ((rows0, sg0, sw0), (rows1, sg1, sw1))

            def start_gather(g, rows, sg):
                pltpu.async_copy(
                    table_hbm.at[idx_v.at[pl.ds(g * C, C)]],
                    rows.at[pl.ds(0, C)], sg)

            def wait_gather(rows, sg):
                pltpu.make_async_copy(
                    table_hbm.at[idx_v.at[pl.ds(0, C)]],
                    rows.at[pl.ds(0, C)], sg).wait()

            def start_write(g, rows, sw):
                for r in range(rows_per_chunk):
                    pltpu.async_copy(
                        rows.at[pl.ds(r * L, L)],
                        out_hbm.at[b_base + g * rows_per_chunk + r]
                               .at[pl.ds(0, L), pl.ds(0, D)], sw)

            def wait_write(rows, sw):
                for r in range(rows_per_chunk):
                    pltpu.make_async_copy(
                        rows.at[pl.ds(r * L, L)],
                        out_hbm.at[b_base].at[pl.ds(0, L), pl.ds(0, D)],
                        sw).wait()

            start_gather(0, rows0, sg0)
            start_gather(1, rows1, sg1)

            @pl.loop(0, n, step=2)
            def _(g):
                for j, (rows, sg, sw) in enumerate(bufs):
                    gg = g + j
                    wait_gather(rows, sg)
                    start_write(gg, rows, sw)

                    @pl.when(gg + 2 < n)
                    def _():
                        wait_write(rows, sw)
                        start_gather(gg + 2, rows, sg)

            wait_write(rows0, sw0)
            wait_write(rows1, sw1)

        pipe(ctx_idx_hbm, ctx_out, ctx_per_w, CL, 1)
        pipe(q_idx_hbm, q_out, q_per_w, QL, 2)

    return k(table, ctx_idx, q_idx)


def kernel(word_embeddings, input_context, input_question):
    B, CL = input_context.shape
    _, QL = input_question.shape
    D = word_embeddings.shape[1]
    QLP = (QL + 7) // 8 * 8
    ctx_idx = input_context.reshape(-1).astype(jnp.int32)
    q_idx = input_question.reshape(-1).astype(jnp.int32)
    ctx_pad, q_pad = _sc_gather(word_embeddings, ctx_idx, q_idx,
                                B, CL, QL, QLP)
    return (ctx_pad[:, :, :D], q_pad[:, :QL, :D])


# speedup vs baseline: 1.3366x; 1.0032x over previous
import functools

import jax
import jax.numpy as jnp
from jax import lax
from jax.experimental import pallas as pl
from jax.experimental.pallas import tpu as pltpu
from jax.experimental.pallas import tpu_sc as plsc

NC = 2
NS = 16
NW = NC * NS


def _sc_gather(table, ctx_idx, q_idx, B, CL, QL, QLP):
    V, D = table.shape
    DP = 2 * D
    b_per_w = B // NW
    ctx_per_w = b_per_w * CL
    q_per_w = b_per_w * QL

    mesh = plsc.VectorSubcoreMesh(core_axis_name="c", subcore_axis_name="s")

    @functools.partial(
        pl.kernel,
        mesh=mesh,
        compiler_params=pltpu.CompilerParams(use_tc_tiling_on_sc=False),
        out_type=(
            jax.ShapeDtypeStruct((B, CL, DP), jnp.float32),
            jax.ShapeDtypeStruct((B, QLP, DP), jnp.float32),
        ),
        scratch_types=[
            pltpu.VMEM((ctx_per_w,), jnp.int32),
            pltpu.VMEM((2 * CL, D), jnp.float32),
            pltpu.VMEM((2 * CL, D), jnp.float32),
            pltpu.SemaphoreType.DMA,
            pltpu.SemaphoreType.DMA,
            pltpu.SemaphoreType.DMA,
            pltpu.SemaphoreType.DMA,
        ],
    )
    def k(table_hbm, ctx_idx_hbm, q_idx_hbm, ctx_out, q_out,
          idx_v, rows0, rows1, sg0, sg1, sw0, sw1):
        wid = lax.axis_index("s") * NC + lax.axis_index("c")
        b_base = wid * b_per_w

        def pipe(idx_hbm, out_hbm, per_w, L, rows_per_chunk):
            C = L * rows_per_chunk
            n = b_per_w // rows_per_chunk
            base = wid * per_w
            pltpu.sync_copy(idx_hbm.at[pl.ds(base, per_w)],
                            idx_v.at[pl.ds(0, per_w)])
            bufs = ((rows0, sg0, sw0), (rows1, sg1, sw1))

            def start_gather(g, rows, sg):
                pltpu.async_copy(
                    table_hbm.at[idx_v.at[pl.ds(g * C, C)]],
                    rows.at[pl.ds(0, C)], sg)

            def wait_gather(rows, sg):
                pltpu.make_async_copy(
                    table_hbm.at[idx_v.at[pl.ds(0, C)]],
                    rows.at[pl.ds(0, C)], sg).wait()

            def start_write(g, rows, sw):
                for r in range(rows_per_chunk):
                    pltpu.async_copy(
                        rows.at[pl.ds(r * L, L)],
                        out_hbm.at[b_base + g * rows_per_chunk + r]
                               .at[pl.ds(0, L), pl.ds(0, D)], sw)

            def wait_write(rows, sw):
                for r in range(rows_per_chunk):
                    pltpu.make_async_copy(
                        rows.at[pl.ds(r * L, L)],
                        out_hbm.at[b_base].at[pl.ds(0, L), pl.ds(0, D)],
                        sw).wait()

            start_gather(0, rows0, sg0)
            start_gather(1, rows1, sg1)

            @pl.loop(0, n, step=2)
            def _(g):
                for j, (rows, sg, sw) in enumerate(bufs):
                    gg = g + j
                    wait_gather(rows, sg)
                    start_write(gg, rows, sw)

                    @pl.when(gg + 2 < n)
                    def _():
                        wait_write(rows, sw)
                        start_gather(gg + 2, rows, sg)

            wait_write(rows0, sw0)
            wait_write(rows1, sw1)

        pipe(ctx_idx_hbm, ctx_out, ctx_per_w, CL, 2)
        pipe(q_idx_hbm, q_out, q_per_w, QL, 2)

    return k(table, ctx_idx, q_idx)


def kernel(word_embeddings, input_context, input_question):
    B, CL = input_context.shape
    _, QL = input_question.shape
    D = word_embeddings.shape[1]
    QLP = (QL + 7) // 8 * 8
    ctx_idx = input_context.reshape(-1).astype(jnp.int32)
    q_idx = input_question.reshape(-1).astype(jnp.int32)
    ctx_pad, q_pad = _sc_gather(word_embeddings, ctx_idx, q_idx,
                                B, CL, QL, QLP)
    return (ctx_pad[:, :, :D], q_pad[:, :QL, :D])
